# baseline (speedup 1.0000x reference)
import functools

import jax
import jax.numpy as jnp
from jax.experimental import pallas as pl
from jax.experimental.pallas import tpu as pltpu

B, N, D_FEAT, D_INNER, D_ATT = 4, 4096, 1024, 1024, 128
N1, N2, N3, N_CLASS = 1, 2, 2, 2
TN = 512
NT = N // TN


def _stage1_body(x_ref, w1_ref, vw_ref, vb_ref, uw_ref, ub_ref, aw_ref, ab_ref,
                 logits_ref, afeat_ref, m_ref, s_ref, acc_ref):
    t = pl.program_id(1)

    feat = x_ref[0, 0]
    xr = jax.nn.relu(
        jax.lax.dot_general(feat, w1_ref[...], (((1,), (1,)), ((), ())),
                            preferred_element_type=jnp.float32, precision=jax.lax.Precision.HIGHEST))
    av = jnp.tanh(
        jax.lax.dot_general(xr, vw_ref[...], (((1,), (1,)), ((), ())),
                            preferred_element_type=jnp.float32, precision=jax.lax.Precision.HIGHEST) + vb_ref[...])
    au = jax.nn.sigmoid(
        jax.lax.dot_general(xr, uw_ref[...], (((1,), (1,)), ((), ())),
                            preferred_element_type=jnp.float32, precision=jax.lax.Precision.HIGHEST) + ub_ref[...])
    h = av * au
    l = jax.lax.dot_general(aw_ref[...], h, (((1,), (1,)), ((), ())),
                            preferred_element_type=jnp.float32, precision=jax.lax.Precision.HIGHEST) + ab_ref[...]
    logits_ref[0, 0:1, pl.ds(t * TN, TN)] = l

    m_t = jnp.max(l, axis=1, keepdims=True)

    @pl.when(t == 0)
    def _():
        m_ref[...] = jnp.full_like(m_ref, -jnp.inf)
        s_ref[...] = jnp.zeros_like(s_ref)
        acc_ref[...] = jnp.zeros_like(acc_ref)

    m_old = m_ref[...]
    m_new = jnp.maximum(m_old, m_t)
    corr = jnp.exp(m_old - m_new)
    p = jnp.exp(l - m_new)
    s_ref[...] = s_ref[...] * corr + jnp.sum(p, axis=1, keepdims=True)
    acc_ref[...] = acc_ref[...] * corr + jax.lax.dot_general(
        p, feat, (((1,), (0,)), ((), ())), preferred_element_type=jnp.float32, precision=jax.lax.Precision.HIGHEST)
    m_ref[...] = m_new

    @pl.when(t == NT - 1)
    def _():
        afeat_ref[0] = acc_ref[...] / s_ref[...]


def _stage2_body(logits_ref, afeat_ref, w2_ref,
                 a2vw_ref, a2vb_ref, a2uw_ref, a2ub_ref, a2w_ref, a2b_ref,
                 a3vw_ref, a3vb_ref, a3uw_ref, a3ub_ref, a3w_ref, a3b_ref,
                 clsw0_ref, clsw1_ref, clsb_ref, slw_ref, slb_ref,
                 a1_ref, a2_ref, out_ref, slide_ref):
    l = logits_ref[...]
    m = jnp.max(l, axis=1, keepdims=True)
    p = jnp.exp(l - m)
    a1_ref[...] = p / jnp.sum(p, axis=1, keepdims=True)

    afeat = afeat_ref[...]
    y = jax.nn.relu(
        jax.lax.dot_general(afeat, w2_ref[...], (((1,), (1,)), ((), ())),
                            preferred_element_type=jnp.float32, precision=jax.lax.Precision.HIGHEST))
    av = jnp.tanh(
        jax.lax.dot_general(y, a2vw_ref[...], (((1,), (1,)), ((), ())),
                            preferred_element_type=jnp.float32, precision=jax.lax.Precision.HIGHEST) + a2vb_ref[...])
    au = jax.nn.sigmoid(
        jax.lax.dot_general(y, a2uw_ref[...], (((1,), (1,)), ((), ())),
                            preferred_element_type=jnp.float32, precision=jax.lax.Precision.HIGHEST) + a2ub_ref[...])
    h = av * au
    a2p = jax.lax.dot_general(a2w_ref[...], h, (((1,), (1,)), ((), ())),
                              preferred_element_type=jnp.float32, precision=jax.lax.Precision.HIGHEST) + a2b_ref[...]
    m2 = jnp.max(a2p, axis=1, keepdims=True)
    e2 = jnp.exp(a2p - m2)
    a2 = e2 / jnp.sum(e2, axis=1, keepdims=True)
    a2_ref[...] = a2

    afeat2 = jax.lax.dot_general(a2, afeat, (((1,), (0,)), ((), ())),
                                 preferred_element_type=jnp.float32, precision=jax.lax.Precision.HIGHEST)
    o0 = jnp.sum(afeat2 * clsw0_ref[...], axis=1, keepdims=True)
    o1 = jnp.sum(afeat2 * clsw1_ref[...], axis=1, keepdims=True)
    out_ref[...] = jnp.concatenate([o0, o1], axis=1) + clsb_ref[...]

    ii = jax.lax.broadcasted_iota(jnp.int32, (2, N2 + N3), 0)
    jj = jax.lax.broadcasted_iota(jnp.int32, (2, N2 + N3), 1)
    mix = jnp.where(((ii == 0) & (jj >= N2)) | ((ii == 1) & (jj < N2)),
                    0.5, 0.0)
    bag_a = jax.lax.dot_general(mix, a2, (((1,), (0,)), ((), ())),
                                preferred_element_type=jnp.float32, precision=jax.lax.Precision.HIGHEST)
    bag_feat = jax.lax.dot_general(bag_a, afeat, (((1,), (0,)), ((), ())),
                                   preferred_element_type=jnp.float32, precision=jax.lax.Precision.HIGHEST)
    av3 = jnp.tanh(
        jax.lax.dot_general(bag_feat, a3vw_ref[...], (((1,), (1,)), ((), ())),
                            preferred_element_type=jnp.float32, precision=jax.lax.Precision.HIGHEST) + a3vb_ref[...])
    au3 = jax.nn.sigmoid(
        jax.lax.dot_general(bag_feat, a3uw_ref[...], (((1,), (1,)), ((), ())),
                            preferred_element_type=jnp.float32, precision=jax.lax.Precision.HIGHEST) + a3ub_ref[...])
    h3 = av3 * au3
    a3p = jax.lax.dot_general(a3w_ref[...], h3, (((1,), (1,)), ((), ())),
                              preferred_element_type=jnp.float32, precision=jax.lax.Precision.HIGHEST) + a3b_ref[...]
    m3 = jnp.max(a3p, axis=1, keepdims=True)
    e3 = jnp.exp(a3p - m3)
    a3 = e3 / jnp.sum(e3, axis=1, keepdims=True)
    fb = jax.lax.dot_general(a3, bag_feat, (((1,), (0,)), ((), ())),
                             preferred_element_type=jnp.float32, precision=jax.lax.Precision.HIGHEST)
    slide_ref[...] = jax.lax.dot_general(
        fb, slw_ref[...], (((1,), (1,)), ((), ())),
        preferred_element_type=jnp.float32, precision=jax.lax.Precision.HIGHEST) + slb_ref[...]


@functools.partial(jax.jit, static_argnums=())
def kernel(x, W_dr1, W_dr2, a1_Vw, a1_Vb, a1_Uw, a1_Ub, a1_w, a1_b,
           a2_Vw, a2_Vb, a2_Uw, a2_Ub, a2_w, a2_b,
           a3_Vw, a3_Vb, a3_Uw, a3_Ub, a3_w, a3_b,
           cls_w, cls_b, slide_w, slide_b):
    f32 = jnp.float32
    whole = lambda shape: pl.BlockSpec(shape, lambda b, t: (0,) * len(shape))

    logits, afeat = pl.pallas_call(
        _stage1_body,
        grid=(B, NT),
        in_specs=[
            pl.BlockSpec((1, 1, TN, D_FEAT), lambda b, t: (0, b, t, 0)),
            whole((D_INNER, D_FEAT)),
            whole((D_ATT, D_INNER)),
            whole((1, D_ATT)),
            whole((D_ATT, D_INNER)),
            whole((1, D_ATT)),
            whole((N1, D_ATT)),
            whole((1, N1)),
        ],
        out_specs=[
            pl.BlockSpec((1, 1, N), lambda b, t: (b, 0, 0)),
            pl.BlockSpec((1, 1, D_FEAT), lambda b, t: (b, 0, 0)),
        ],
        out_shape=[
            jax.ShapeDtypeStruct((B, 1, N), f32),
            jax.ShapeDtypeStruct((B, 1, D_FEAT), f32),
        ],
        scratch_shapes=[
            pltpu.VMEM((1, 1), f32),
            pltpu.VMEM((1, 1), f32),
            pltpu.VMEM((1, D_FEAT), f32),
        ],
        compiler_params=pltpu.CompilerParams(
            dimension_semantics=("arbitrary", "arbitrary")),
    )(x, W_dr1, a1_Vw, a1_Vb.reshape(1, D_ATT), a1_Uw,
      a1_Ub.reshape(1, D_ATT), a1_w, a1_b.reshape(1, N1))

    logits = logits.reshape(B, N)
    afeat = afeat.reshape(B, D_FEAT)

    T = N2 + N3
    a1_out, a2_out, outputs, slide = pl.pallas_call(
        _stage2_body,
        out_shape=[
            jax.ShapeDtypeStruct((B, N), f32),
            jax.ShapeDtypeStruct((T, B), f32),
            jax.ShapeDtypeStruct((T, N_CLASS), f32),
            jax.ShapeDtypeStruct((1, N_CLASS), f32),
        ],
    )(logits, afeat, W_dr2,
      a2_Vw, a2_Vb.reshape(1, D_ATT), a2_Uw, a2_Ub.reshape(1, D_ATT),
      a2_w, a2_b.reshape(T, 1),
      a3_Vw, a3_Vb.reshape(1, D_ATT), a3_Uw, a3_Ub.reshape(1, D_ATT),
      a3_w, a3_b.reshape(1, 1),
      cls_w[:, 0, :], cls_w[:, 1, :], cls_b, slide_w, slide_b.reshape(1, N_CLASS))

    A_1 = a1_out.reshape(B, N1, N)
    return (outputs, slide, A_1, a2_out[:N2], a2_out[N2:], a2_out)


# speedup vs baseline: 2.1754x; 2.1754x over previous
import functools

import jax
import jax.numpy as jnp
from jax.experimental import pallas as pl
from jax.experimental.pallas import tpu as pltpu

B, N, D_FEAT, D_INNER, D_ATT = 4, 4096, 1024, 1024, 128
N1, N2, N3, N_CLASS = 1, 2, 2, 2
TN = 512
NT = N // TN


def _dot_t(a, b):
    return jax.lax.dot_general(a, b, (((1,), (1,)), ((), ())),
                               preferred_element_type=jnp.float32)


def _stage1_body(x_ref, w1h_ref, w1l_ref, vw_ref, vb_ref, uw_ref, ub_ref,
                 aw_ref, ab_ref, logits_ref, afeat_ref, m_ref, s_ref, acc_ref):
    t = pl.program_id(1)

    feat = x_ref[0, 0]
    f_hi = feat.astype(jnp.bfloat16)
    f_lo = (feat - f_hi.astype(jnp.float32)).astype(jnp.bfloat16)
    xr = jax.nn.relu(
        _dot_t(f_hi, w1h_ref[...]) + _dot_t(f_hi, w1l_ref[...])
        + _dot_t(f_lo, w1h_ref[...]))
    av = jnp.tanh(
        jax.lax.dot_general(xr, vw_ref[...], (((1,), (1,)), ((), ())),
                            preferred_element_type=jnp.float32) + vb_ref[...])
    au = jax.nn.sigmoid(
        jax.lax.dot_general(xr, uw_ref[...], (((1,), (1,)), ((), ())),
                            preferred_element_type=jnp.float32) + ub_ref[...])
    h = av * au
    l = jax.lax.dot_general(aw_ref[...], h, (((1,), (1,)), ((), ())),
                            preferred_element_type=jnp.float32) + ab_ref[...]
    logits_ref[0, 0:1, pl.ds(t * TN, TN)] = l

    m_t = jnp.max(l, axis=1, keepdims=True)

    @pl.when(t == 0)
    def _():
        m_ref[...] = jnp.full_like(m_ref, -jnp.inf)
        s_ref[...] = jnp.zeros_like(s_ref)
        acc_ref[...] = jnp.zeros_like(acc_ref)

    m_old = m_ref[...]
    m_new = jnp.maximum(m_old, m_t)
    corr = jnp.exp(m_old - m_new)
    p = jnp.exp(l - m_new)
    s_ref[...] = s_ref[...] * corr + jnp.sum(p, axis=1, keepdims=True)
    p_bf = p.astype(jnp.bfloat16)
    pooled = (jax.lax.dot_general(p_bf, f_hi, (((1,), (0,)), ((), ())),
                                  preferred_element_type=jnp.float32)
              + jax.lax.dot_general(p_bf, f_lo, (((1,), (0,)), ((), ())),
                                    preferred_element_type=jnp.float32))
    acc_ref[...] = acc_ref[...] * corr + pooled
    m_ref[...] = m_new

    @pl.when(t == NT - 1)
    def _():
        afeat_ref[0] = acc_ref[...] / s_ref[...]


def _stage2_body(logits_ref, afeat_ref, w2_ref,
                 a2vw_ref, a2vb_ref, a2uw_ref, a2ub_ref, a2w_ref, a2b_ref,
                 a3vw_ref, a3vb_ref, a3uw_ref, a3ub_ref, a3w_ref, a3b_ref,
                 clsw0_ref, clsw1_ref, clsb_ref, slw_ref, slb_ref,
                 a1_ref, a2_ref, out_ref, slide_ref):
    l = logits_ref[...]
    m = jnp.max(l, axis=1, keepdims=True)
    p = jnp.exp(l - m)
    a1_ref[...] = p / jnp.sum(p, axis=1, keepdims=True)

    afeat = afeat_ref[...]
    y = jax.nn.relu(
        jax.lax.dot_general(afeat, w2_ref[...], (((1,), (1,)), ((), ())),
                            preferred_element_type=jnp.float32, precision=jax.lax.Precision.HIGHEST))
    av = jnp.tanh(
        jax.lax.dot_general(y, a2vw_ref[...], (((1,), (1,)), ((), ())),
                            preferred_element_type=jnp.float32, precision=jax.lax.Precision.HIGHEST) + a2vb_ref[...])
    au = jax.nn.sigmoid(
        jax.lax.dot_general(y, a2uw_ref[...], (((1,), (1,)), ((), ())),
                            preferred_element_type=jnp.float32, precision=jax.lax.Precision.HIGHEST) + a2ub_ref[...])
    h = av * au
    a2p = jax.lax.dot_general(a2w_ref[...], h, (((1,), (1,)), ((), ())),
                              preferred_element_type=jnp.float32, precision=jax.lax.Precision.HIGHEST) + a2b_ref[...]
    m2 = jnp.max(a2p, axis=1, keepdims=True)
    e2 = jnp.exp(a2p - m2)
    a2 = e2 / jnp.sum(e2, axis=1, keepdims=True)
    a2_ref[...] = a2

    afeat2 = jax.lax.dot_general(a2, afeat, (((1,), (0,)), ((), ())),
                                 preferred_element_type=jnp.float32, precision=jax.lax.Precision.HIGHEST)
    o0 = jnp.sum(afeat2 * clsw0_ref[...], axis=1, keepdims=True)
    o1 = jnp.sum(afeat2 * clsw1_ref[...], axis=1, keepdims=True)
    out_ref[...] = jnp.concatenate([o0, o1], axis=1) + clsb_ref[...]

    ii = jax.lax.broadcasted_iota(jnp.int32, (2, N2 + N3), 0)
    jj = jax.lax.broadcasted_iota(jnp.int32, (2, N2 + N3), 1)
    mix = jnp.where(((ii == 0) & (jj >= N2)) | ((ii == 1) & (jj < N2)),
                    0.5, 0.0)
    bag_a = jax.lax.dot_general(mix, a2, (((1,), (0,)), ((), ())),
                                preferred_element_type=jnp.float32, precision=jax.lax.Precision.HIGHEST)
    bag_feat = jax.lax.dot_general(bag_a, afeat, (((1,), (0,)), ((), ())),
                                   preferred_element_type=jnp.float32, precision=jax.lax.Precision.HIGHEST)
    av3 = jnp.tanh(
        jax.lax.dot_general(bag_feat, a3vw_ref[...], (((1,), (1,)), ((), ())),
                            preferred_element_type=jnp.float32, precision=jax.lax.Precision.HIGHEST) + a3vb_ref[...])
    au3 = jax.nn.sigmoid(
        jax.lax.dot_general(bag_feat, a3uw_ref[...], (((1,), (1,)), ((), ())),
                            preferred_element_type=jnp.float32, precision=jax.lax.Precision.HIGHEST) + a3ub_ref[...])
    h3 = av3 * au3
    a3p = jax.lax.dot_general(a3w_ref[...], h3, (((1,), (1,)), ((), ())),
                              preferred_element_type=jnp.float32, precision=jax.lax.Precision.HIGHEST) + a3b_ref[...]
    m3 = jnp.max(a3p, axis=1, keepdims=True)
    e3 = jnp.exp(a3p - m3)
    a3 = e3 / jnp.sum(e3, axis=1, keepdims=True)
    fb = jax.lax.dot_general(a3, bag_feat, (((1,), (0,)), ((), ())),
                             preferred_element_type=jnp.float32, precision=jax.lax.Precision.HIGHEST)
    slide_ref[...] = jax.lax.dot_general(
        fb, slw_ref[...], (((1,), (1,)), ((), ())),
        preferred_element_type=jnp.float32, precision=jax.lax.Precision.HIGHEST) + slb_ref[...]


@functools.partial(jax.jit, static_argnums=())
def kernel(x, W_dr1, W_dr2, a1_Vw, a1_Vb, a1_Uw, a1_Ub, a1_w, a1_b,
           a2_Vw, a2_Vb, a2_Uw, a2_Ub, a2_w, a2_b,
           a3_Vw, a3_Vb, a3_Uw, a3_Ub, a3_w, a3_b,
           cls_w, cls_b, slide_w, slide_b):
    f32 = jnp.float32
    whole = lambda shape: pl.BlockSpec(shape, lambda b, t: (0,) * len(shape))

    stage1 = pl.pallas_call(
        _stage1_body,
        grid=(B, NT),
        in_specs=[
            pl.BlockSpec((1, 1, TN, D_FEAT), lambda b, t: (0, b, t, 0)),
            whole((D_INNER, D_FEAT)),
            whole((D_INNER, D_FEAT)),
            whole((D_ATT, D_INNER)),
            whole((1, D_ATT)),
            whole((D_ATT, D_INNER)),
            whole((1, D_ATT)),
            whole((N1, D_ATT)),
            whole((1, N1)),
        ],
        out_specs=[
            pl.BlockSpec((1, 1, N), lambda b, t: (b, 0, 0)),
            pl.BlockSpec((1, 1, D_FEAT), lambda b, t: (b, 0, 0)),
        ],
        out_shape=[
            jax.ShapeDtypeStruct((B, 1, N), f32),
            jax.ShapeDtypeStruct((B, 1, D_FEAT), f32),
        ],
        scratch_shapes=[
            pltpu.VMEM((1, 1), f32),
            pltpu.VMEM((1, 1), f32),
            pltpu.VMEM((1, D_FEAT), f32),
        ],
        compiler_params=pltpu.CompilerParams(
            dimension_semantics=("arbitrary", "arbitrary")),
    )
    w1_hi = W_dr1.astype(jnp.bfloat16)
    w1_lo = (W_dr1 - w1_hi.astype(f32)).astype(jnp.bfloat16)
    logits, afeat = stage1(x, w1_hi, w1_lo, a1_Vw, a1_Vb.reshape(1, D_ATT),
                           a1_Uw, a1_Ub.reshape(1, D_ATT), a1_w,
                           a1_b.reshape(1, N1))

    logits = logits.reshape(B, N)
    afeat = afeat.reshape(B, D_FEAT)

    T = N2 + N3
    a1_out, a2_out, outputs, slide = pl.pallas_call(
        _stage2_body,
        out_shape=[
            jax.ShapeDtypeStruct((B, N), f32),
            jax.ShapeDtypeStruct((T, B), f32),
            jax.ShapeDtypeStruct((T, N_CLASS), f32),
            jax.ShapeDtypeStruct((1, N_CLASS), f32),
        ],
    )(logits, afeat, W_dr2,
      a2_Vw, a2_Vb.reshape(1, D_ATT), a2_Uw, a2_Ub.reshape(1, D_ATT),
      a2_w, a2_b.reshape(T, 1),
      a3_Vw, a3_Vb.reshape(1, D_ATT), a3_Uw, a3_Ub.reshape(1, D_ATT),
      a3_w, a3_b.reshape(1, 1),
      cls_w[:, 0, :], cls_w[:, 1, :], cls_b, slide_w, slide_b.reshape(1, N_CLASS))

    A_1 = a1_out.reshape(B, N1, N)
    return (outputs, slide, A_1, a2_out[:N2], a2_out[N2:], a2_out)
